# baseline (speedup 1.0000x reference)
import functools

import jax
import jax.numpy as jnp
from jax import lax
from jax.experimental import pallas as pl
from jax.experimental.pallas import tpu as pltpu
from jax.experimental.pallas import tpu_sc as plsc

B = 16384
DIM = 32
NCH = 8
SIZE = 16777216
LANES = 16
TROWS = SIZE // LANES
RMASK = TROWS - 1
PRIME = (1 << 31) - 1

NC, NS = 2, 16
NW = NC * NS
RPW = B // NW
NB = 16
NBLK = RPW // NB
LPB = NB * NCH


def _fold(s):
    return (s & jnp.uint32(PRIME)) + (s >> 31)


def _rot(n, k):
    low = (n & jnp.uint32((1 << (31 - k)) - 1)) << k
    high = n >> (31 - k)
    return low + high


def _hash(x1, x0, a1, a0, bb):
    s = _fold(_rot(a1 * x1, 26) + a0 * x0)
    s = _fold(s + _rot(a1 * x0, 16))
    s = _fold(s + _rot(a0 * x1, 10))
    s = _fold(s + bb)
    s = _fold(s)
    s = jnp.where(s >= jnp.uint32(PRIME), s - jnp.uint32(PRIME), s)
    return s & jnp.uint32(SIZE - 1)


SROWS = 4 * LPB
SFLAT = SROWS * LANES


def _body(x_hbm, tab_hbm, cf_hbm, out_hbm,
          xv, cfv,
          sidx0, sidx1, sidx2, sidx3,
          bv0, bv1, bv2, bv3,
          wb0, wb1, wb2, wb3,
          sstage0, sstage1, sstage2, sstage3,
          outv, sem0, sem1, sem2, sem3):
    wid = lax.axis_index("s") * jnp.int32(NC) + lax.axis_index("c")
    base = wid * jnp.int32(RPW)
    pltpu.sync_copy(x_hbm.at[pl.ds(base, RPW)], xv)
    pltpu.sync_copy(cf_hbm, cfv)
    lanes = lax.iota(jnp.int32, LANES)
    lanes512 = lanes * 512

    cfr = [cfv[pl.ds(r * LANES, LANES)] for r in range(6)]
    coef = [[cfr[r][j] for r in range(6)] for j in range(NCH)]

    def hash_block(b, sidx, bv, wb):
        xu = xv[pl.ds(b * jnp.int32(NB), NB)]
        x1 = xu >> 10
        x0 = xu & jnp.uint32(1023)
        for j in range(NCH):
            a1_0, a0_0, b_0, a1_1, a0_1, b_1 = coef[j]
            h0 = _hash(x1, x0, a1_0, a0_0, b_0)
            r = (h0 >> 4).astype(jnp.int32)
            o0 = (h0 & jnp.uint32(15)).astype(jnp.int32)
            h1 = _hash(x1, x0, a1_1, a0_1, b_1)
            wr = (h1 >> 4).astype(jnp.int32)
            wo = (h1 & jnp.uint32(15)).astype(jnp.int32)
            posb = lanes * 32 + 4 * j
            plsc.store_scatter(sidx, [posb], r)
            plsc.store_scatter(sidx, [posb + 1], (r + 1) & RMASK)
            plsc.store_scatter(sidx, [posb + 2], (r + 2) & RMASK)
            plsc.store_scatter(sidx, [posb + 3], wr)
            slot = lanes * LANES + j
            plsc.store_scatter(bv, [slot], lanes512 + (o0 + 64 * j))
            plsc.store_scatter(wb, [slot], lanes512 + (wo + (64 * j + 48)))

    def copies(sidx, sstage, sem):
        return [
            pltpu.make_async_copy(
                tab_hbm.at[sidx.at[pl.ds(d * LPB, LPB)]],
                sstage.at[pl.ds(d * LPB, LPB)], sem)
            for d in range(4)
        ]

    def fire(bufs):
        for c in copies(*bufs):
            c.start()

    def drain(bufs):
        for c in copies(*bufs):
            c.wait()

    def _tree8(p):
        return ((p[0] + p[1]) + (p[2] + p[3])) + ((p[4] + p[5]) + (p[6] + p[7]))

    def accum(b, bv, wb, sstage):
        for rp in range(NB // 2):
            rows = (2 * rp, 2 * rp + 1)
            bvr, wvr, prods = [], [], []
            for row in rows:
                bvrow = bv[pl.ds(row * LANES, LANES)]
                wposv = wb[pl.ds(row * LANES, LANES)]
                wp = wposv & (SFLAT - 1)
                bvr.append(bvrow)
                wvr.append(plsc.load_gather(sstage, [wp >> 4, wp & 15]))
                prods.append(([], []))
            for j in range(NCH):
                for t in range(2):
                    f0 = bvr[t][j] + lanes
                    r0 = f0 >> 4
                    e0 = f0 & 15
                    g0 = plsc.load_gather(sstage, [r0, e0])
                    g1 = plsc.load_gather(sstage, [r0 + 1, e0])
                    w = wvr[t][j]
                    prods[t][0].append(g0 * w)
                    prods[t][1].append(g1 * w)
            for t in range(2):
                orow = b * jnp.int32(NB) + rows[t]
                outv[orow, pl.ds(0, LANES)] = _tree8(prods[t][0]) * 2.0
                outv[orow, pl.ds(LANES, LANES)] = _tree8(prods[t][1]) * 2.0

    sets = [
        (sidx0, bv0, wb0, sstage0, sem0),
        (sidx1, bv1, wb1, sstage1, sem1),
        (sidx2, bv2, wb2, sstage2, sem2),
        (sidx3, bv3, wb3, sstage3, sem3),
    ]

    def hash_fire(b, s):
        sidx, bvb, wbb, sstage, sem = s
        hash_block(b, sidx, bvb, wbb)
        fire((sidx, sstage, sem))

    def drain_accum(b, s):
        sidx, bvb, wbb, sstage, sem = s
        drain((sidx, sstage, sem))
        accum(b, bvb, wbb, sstage)

    hash_fire(jnp.int32(0), sets[0])
    hash_fire(jnp.int32(1), sets[1])

    def quad(i, carry):
        b = i * jnp.int32(4)
        hash_fire(b + 2, sets[2])
        drain_accum(b, sets[0])
        hash_fire(b + 3, sets[3])
        drain_accum(b + 1, sets[1])

        @pl.when(i < jnp.int32(NBLK // 4 - 1))
        def _():
            hash_fire(b + 4, sets[0])

        drain_accum(b + 2, sets[2])

        @pl.when(i < jnp.int32(NBLK // 4 - 1))
        def _():
            hash_fire(b + 5, sets[1])

        drain_accum(b + 3, sets[3])
        return carry

    lax.fori_loop(jnp.int32(0), jnp.int32(NBLK // 4), quad, jnp.int32(0))
    pltpu.sync_copy(outv, out_hbm.at[pl.ds(base, RPW)])


@jax.jit
def _sc_call(xs, tab2d, cf):
    mesh = plsc.VectorSubcoreMesh(core_axis_name="c", subcore_axis_name="s")
    f = functools.partial(
        pl.kernel,
        out_type=jax.ShapeDtypeStruct((B, DIM), jnp.float32),
        mesh=mesh,
        scratch_types=[
            pltpu.VMEM((RPW,), jnp.uint32),
            pltpu.VMEM((6 * LANES,), jnp.uint32),
            *[pltpu.VMEM((4 * LPB,), jnp.int32) for _ in range(4)],
            *[pltpu.VMEM((NB * LANES,), jnp.int32) for _ in range(4)],
            *[pltpu.VMEM((NB * LANES,), jnp.int32) for _ in range(4)],
            *[pltpu.VMEM((SROWS, LANES), jnp.float32) for _ in range(4)],
            pltpu.VMEM((RPW, DIM), jnp.float32),
            *[pltpu.SemaphoreType.DMA for _ in range(4)],
        ],
        compiler_params=pltpu.CompilerParams(
            needs_layout_passes=False, use_tc_tiling_on_sc=False),
    )(_body)
    return f(xs, tab2d, cf)


def kernel(x, table0, coeffs0, coeffs1):
    xs = x.astype(jnp.uint32)
    tab2d = table0.reshape(TROWS, LANES)

    def split(c):
        a = c[:, 0]
        return jnp.stack([a >> 16, a & 0xFFFF, c[:, 1]])

    cf = jnp.concatenate([split(coeffs0), split(coeffs1)]).astype(jnp.uint32)
    cf = jnp.pad(cf, ((0, 0), (0, LANES - NCH))).reshape(-1)
    return _sc_call(xs, tab2d, cf)


# speedup vs baseline: 1.2072x; 1.2072x over previous
import functools

import jax
import jax.numpy as jnp
from jax import lax
from jax.experimental import pallas as pl
from jax.experimental.pallas import tpu as pltpu
from jax.experimental.pallas import tpu_sc as plsc

B = 16384
DIM = 32
NCH = 8
SIZE = 16777216
LANES = 16
TROWS = SIZE // LANES
RMASK = TROWS - 1
PRIME = (1 << 31) - 1

NC, NS = 2, 16
NW = NC * NS
RPW = B // NW
NB = 16
NBLK = RPW // NB
LPB = NB * NCH


def _fold(s):
    return (s & jnp.uint32(PRIME)) + (s >> 31)


def _rot(n, k):
    low = (n & jnp.uint32((1 << (31 - k)) - 1)) << k
    high = n >> (31 - k)
    return low + high


def _hash_many(x1, x0, cos):
    t3 = [_rot(a1 * x1, 26) for (a1, a0, bb) in cos]
    t0 = [a0 * x0 for (a1, a0, bb) in cos]
    t2 = [_rot(a1 * x0, 16) for (a1, a0, bb) in cos]
    t1 = [_rot(a0 * x1, 10) for (a1, a0, bb) in cos]
    ss = [_fold(a + b) for a, b in zip(t3, t0)]
    ss = [_fold(s + t) for s, t in zip(ss, t2)]
    ss = [_fold(s + t) for s, t in zip(ss, t1)]
    ss = [_fold(s + bb) for s, (a1, a0, bb) in zip(ss, cos)]
    ss = [_fold(s) for s in ss]
    ss = [jnp.where(s >= jnp.uint32(PRIME), s - jnp.uint32(PRIME), s)
          for s in ss]
    return [s & jnp.uint32(SIZE - 1) for s in ss]


SROWS = 4 * LPB
SFLAT = SROWS * LANES


def _body(x_hbm, tab_hbm, cf_hbm, out_hbm,
          xv, cfv, sidxA, sidxB, bvA, bvB, wbA, wbB,
          sstageA, sstageB, outv, semA, semB):
    wid = lax.axis_index("s") * jnp.int32(NC) + lax.axis_index("c")
    base = wid * jnp.int32(RPW)
    pltpu.sync_copy(x_hbm.at[pl.ds(base, RPW)], xv)
    pltpu.sync_copy(cf_hbm, cfv)
    lanes = lax.iota(jnp.int32, LANES)
    lanes512 = lanes * 512

    cfr = [cfv[pl.ds(r * LANES, LANES)] for r in range(6)]
    coef = [[cfr[r][j] for r in range(6)] for j in range(NCH)]

    def hash_block(b, sidx, bv, wb):
        xu = xv[pl.ds(b * jnp.int32(NB), NB)]
        x1 = xu >> 10
        x0 = xu & jnp.uint32(1023)
        for jp in range(NCH // 2):
            js = (2 * jp, 2 * jp + 1)
            cos = []
            for j in js:
                a1_0, a0_0, b_0, a1_1, a0_1, b_1 = coef[j]
                cos.append((a1_0, a0_0, b_0))
                cos.append((a1_1, a0_1, b_1))
            hs = _hash_many(x1, x0, cos)
            for t, j in enumerate(js):
                h0, h1 = hs[2 * t], hs[2 * t + 1]
                r = (h0 >> 4).astype(jnp.int32)
                o0 = (h0 & jnp.uint32(15)).astype(jnp.int32)
                wr = (h1 >> 4).astype(jnp.int32)
                wo = (h1 & jnp.uint32(15)).astype(jnp.int32)
                posb = lanes * 32 + 4 * j
                plsc.store_scatter(sidx, [posb], r)
                plsc.store_scatter(sidx, [posb + 1], (r + 1) & RMASK)
                plsc.store_scatter(sidx, [posb + 2], (r + 2) & RMASK)
                plsc.store_scatter(sidx, [posb + 3], wr)
                slot = lanes * LANES + j
                plsc.store_scatter(bv, [slot], lanes512 + (o0 + 64 * j))
                plsc.store_scatter(wb, [slot], lanes512 + (wo + (64 * j + 48)))

    def copies(sidx, sstage, sem):
        return [
            pltpu.make_async_copy(
                tab_hbm.at[sidx.at[pl.ds(d * LPB, LPB)]],
                sstage.at[pl.ds(d * LPB, LPB)], sem)
            for d in range(4)
        ]

    def fire(bufs):
        for c in copies(*bufs):
            c.start()

    def drain(bufs):
        for c in copies(*bufs):
            c.wait()

    def _tree8(p):
        return ((p[0] + p[1]) + (p[2] + p[3])) + ((p[4] + p[5]) + (p[6] + p[7]))

    def accum(b, bv, wb, sstage):
        for rp in range(NB // 2):
            rows = (2 * rp, 2 * rp + 1)
            bvr, wvr, prods = [], [], []
            for row in rows:
                bvrow = bv[pl.ds(row * LANES, LANES)]
                wposv = wb[pl.ds(row * LANES, LANES)]
                wp = wposv & (SFLAT - 1)
                bvr.append(bvrow)
                wvr.append(plsc.load_gather(sstage, [wp >> 4, wp & 15]))
                prods.append(([], []))
            for j in range(NCH):
                for t in range(2):
                    f0 = bvr[t][j] + lanes
                    r0 = f0 >> 4
                    e0 = f0 & 15
                    g0 = plsc.load_gather(sstage, [r0, e0])
                    g1 = plsc.load_gather(sstage, [r0 + 1, e0])
                    w = wvr[t][j]
                    prods[t][0].append(g0 * w)
                    prods[t][1].append(g1 * w)
            for t in range(2):
                orow = b * jnp.int32(NB) + rows[t]
                outv[orow, pl.ds(0, LANES)] = _tree8(prods[t][0]) * 2.0
                outv[orow, pl.ds(LANES, LANES)] = _tree8(prods[t][1]) * 2.0

    bufsA = (sidxA, sstageA, semA)
    bufsB = (sidxB, sstageB, semB)

    hash_block(jnp.int32(0), sidxA, bvA, wbA)
    fire(bufsA)

    def pair(i, carry):
        b0 = i * jnp.int32(2)
        b1 = b0 + 1
        hash_block(b1, sidxB, bvB, wbB)
        fire(bufsB)
        drain(bufsA)
        accum(b0, bvA, wbA, sstageA)

        @pl.when(i < jnp.int32(NBLK // 2 - 1))
        def _():
            hash_block(b0 + 2, sidxA, bvA, wbA)
            fire(bufsA)

        drain(bufsB)
        accum(b1, bvB, wbB, sstageB)
        return carry

    lax.fori_loop(jnp.int32(0), jnp.int32(NBLK // 2), pair, jnp.int32(0))
    pltpu.sync_copy(outv, out_hbm.at[pl.ds(base, RPW)])


@jax.jit
def _sc_call(xs, tab2d, cf):
    mesh = plsc.VectorSubcoreMesh(core_axis_name="c", subcore_axis_name="s")
    f = functools.partial(
        pl.kernel,
        out_type=jax.ShapeDtypeStruct((B, DIM), jnp.float32),
        mesh=mesh,
        scratch_types=[
            pltpu.VMEM((RPW,), jnp.uint32),
            pltpu.VMEM((6 * LANES,), jnp.uint32),
            pltpu.VMEM((4 * LPB,), jnp.int32),
            pltpu.VMEM((4 * LPB,), jnp.int32),
            pltpu.VMEM((NB * LANES,), jnp.int32),
            pltpu.VMEM((NB * LANES,), jnp.int32),
            pltpu.VMEM((NB * LANES,), jnp.int32),
            pltpu.VMEM((NB * LANES,), jnp.int32),
            pltpu.VMEM((SROWS, LANES), jnp.float32),
            pltpu.VMEM((SROWS, LANES), jnp.float32),
            pltpu.VMEM((RPW, DIM), jnp.float32),
            pltpu.SemaphoreType.DMA,
            pltpu.SemaphoreType.DMA,
        ],
        compiler_params=pltpu.CompilerParams(
            needs_layout_passes=False, use_tc_tiling_on_sc=False),
    )(_body)
    return f(xs, tab2d, cf)


def kernel(x, table0, coeffs0, coeffs1):
    xs = x.astype(jnp.uint32)
    tab2d = table0.reshape(TROWS, LANES)

    def split(c):
        a = c[:, 0]
        return jnp.stack([a >> 16, a & 0xFFFF, c[:, 1]])

    cf = jnp.concatenate([split(coeffs0), split(coeffs1)]).astype(jnp.uint32)
    cf = jnp.pad(cf, ((0, 0), (0, LANES - NCH))).reshape(-1)
    return _sc_call(xs, tab2d, cf)


# speedup vs baseline: 1.2383x; 1.0258x over previous
import functools

import jax
import jax.numpy as jnp
from jax import lax
from jax.experimental import pallas as pl
from jax.experimental.pallas import tpu as pltpu
from jax.experimental.pallas import tpu_sc as plsc

B = 16384
DIM = 32
NCH = 8
SIZE = 16777216
LANES = 16
TROWS = SIZE // LANES
RMASK = TROWS - 1
PRIME = (1 << 31) - 1

NC, NS = 2, 16
NW = NC * NS
RPW = B // NW
NB = 16
NBLK = RPW // NB
LPB = NB * NCH


def _fold(s):
    return (s & jnp.uint32(PRIME)) + (s >> 31)


def _rot(n, k):
    low = (n & jnp.uint32((1 << (31 - k)) - 1)) << k
    high = n >> (31 - k)
    return low + high


def _hash_many(x1, x0, cos):
    t3 = [_rot(a1 * x1, 26) for (a1, a0, bb) in cos]
    t0 = [a0 * x0 for (a1, a0, bb) in cos]
    t2 = [_rot(a1 * x0, 16) for (a1, a0, bb) in cos]
    t1 = [_rot(a0 * x1, 10) for (a1, a0, bb) in cos]
    ss = [_fold(a + b) for a, b in zip(t3, t0)]
    ss = [_fold(s + t) for s, t in zip(ss, t2)]
    ss = [_fold(s + t) for s, t in zip(ss, t1)]
    ss = [_fold(s + bb) for s, (a1, a0, bb) in zip(ss, cos)]
    ss = [_fold(s) for s in ss]
    ss = [jnp.where(s >= jnp.uint32(PRIME), s - jnp.uint32(PRIME), s)
          for s in ss]
    return [s & jnp.uint32(SIZE - 1) for s in ss]


SROWS = 4 * LPB
SFLAT = SROWS * LANES


def _body(x_hbm, tab_hbm, cf_hbm, out_hbm,
          xv, cfv, sidxA, sidxB, bvA, bvB, wbA, wbB,
          sstageA, sstageB, outv, semA, semB):
    wid = lax.axis_index("s") * jnp.int32(NC) + lax.axis_index("c")
    base = wid * jnp.int32(RPW)
    pltpu.sync_copy(x_hbm.at[pl.ds(base, RPW)], xv)
    pltpu.sync_copy(cf_hbm, cfv)
    lanes = lax.iota(jnp.int32, LANES)
    lanes512 = lanes * 512

    cfr = [cfv[pl.ds(r * LANES, LANES)] for r in range(6)]
    coef = [[cfr[r][j] for r in range(6)] for j in range(NCH)]

    def hash_block(b, sidx, bv, wb):
        xu = xv[pl.ds(b * jnp.int32(NB), NB)]
        x1 = xu >> 10
        x0 = xu & jnp.uint32(1023)
        for jp in range(NCH // 4):
            js = tuple(4 * jp + u for u in range(4))
            cos = []
            for j in js:
                a1_0, a0_0, b_0, a1_1, a0_1, b_1 = coef[j]
                cos.append((a1_0, a0_0, b_0))
                cos.append((a1_1, a0_1, b_1))
            hs = _hash_many(x1, x0, cos)
            for t, j in enumerate(js):
                h0, h1 = hs[2 * t], hs[2 * t + 1]
                r = (h0 >> 4).astype(jnp.int32)
                o0 = (h0 & jnp.uint32(15)).astype(jnp.int32)
                wr = (h1 >> 4).astype(jnp.int32)
                wo = (h1 & jnp.uint32(15)).astype(jnp.int32)
                posb = lanes * 32 + 4 * j
                plsc.store_scatter(sidx, [posb], r)
                plsc.store_scatter(sidx, [posb + 1], (r + 1) & RMASK)
                plsc.store_scatter(sidx, [posb + 2], (r + 2) & RMASK)
                plsc.store_scatter(sidx, [posb + 3], wr)
                slot = lanes * LANES + j
                plsc.store_scatter(bv, [slot], lanes512 + (o0 + 64 * j))
                plsc.store_scatter(wb, [slot], lanes512 + (wo + (64 * j + 48)))

    def copies(sidx, sstage, sem):
        return [
            pltpu.make_async_copy(
                tab_hbm.at[sidx.at[pl.ds(d * LPB, LPB)]],
                sstage.at[pl.ds(d * LPB, LPB)], sem)
            for d in range(4)
        ]

    def fire(bufs):
        for c in copies(*bufs):
            c.start()

    def drain(bufs):
        for c in copies(*bufs):
            c.wait()

    def _tree8(p):
        return ((p[0] + p[1]) + (p[2] + p[3])) + ((p[4] + p[5]) + (p[6] + p[7]))

    NR = 4

    def accum(b, bv, wb, sstage):
        for rp in range(NB // NR):
            rows = tuple(NR * rp + u for u in range(NR))
            bvr, wvr = [], []
            acc = [[None, None, None, None] for _ in rows]
            for row in rows:
                bvrow = bv[pl.ds(row * LANES, LANES)]
                wposv = wb[pl.ds(row * LANES, LANES)]
                wp = wposv & (SFLAT - 1)
                bvr.append(bvrow)
                wvr.append(plsc.load_gather(sstage, [wp >> 4, wp & 15]))
            for j in range(NCH):
                par = j & 1
                for t in range(NR):
                    f0 = bvr[t][j] + lanes
                    r0 = f0 >> 4
                    e0 = f0 & 15
                    g0 = plsc.load_gather(sstage, [r0, e0])
                    g1 = plsc.load_gather(sstage, [r0 + 1, e0])
                    w = wvr[t][j]
                    p0, p1 = g0 * w, g1 * w
                    a = acc[t]
                    a[par] = p0 if a[par] is None else a[par] + p0
                    a[2 + par] = p1 if a[2 + par] is None else a[2 + par] + p1
            for t in range(NR):
                orow = b * jnp.int32(NB) + rows[t]
                a = acc[t]
                outv[orow, pl.ds(0, LANES)] = (a[0] + a[1]) * 2.0
                outv[orow, pl.ds(LANES, LANES)] = (a[2] + a[3]) * 2.0

    bufsA = (sidxA, sstageA, semA)
    bufsB = (sidxB, sstageB, semB)

    hash_block(jnp.int32(0), sidxA, bvA, wbA)
    fire(bufsA)

    def pair(i, carry):
        b0 = i * jnp.int32(2)
        b1 = b0 + 1
        hash_block(b1, sidxB, bvB, wbB)
        fire(bufsB)
        drain(bufsA)
        accum(b0, bvA, wbA, sstageA)

        @pl.when(i < jnp.int32(NBLK // 2 - 1))
        def _():
            hash_block(b0 + 2, sidxA, bvA, wbA)
            fire(bufsA)

        drain(bufsB)
        accum(b1, bvB, wbB, sstageB)
        return carry

    lax.fori_loop(jnp.int32(0), jnp.int32(NBLK // 2), pair, jnp.int32(0))
    pltpu.sync_copy(outv, out_hbm.at[pl.ds(base, RPW)])


@jax.jit
def _sc_call(xs, tab2d, cf):
    mesh = plsc.VectorSubcoreMesh(core_axis_name="c", subcore_axis_name="s")
    f = functools.partial(
        pl.kernel,
        out_type=jax.ShapeDtypeStruct((B, DIM), jnp.float32),
        mesh=mesh,
        scratch_types=[
            pltpu.VMEM((RPW,), jnp.uint32),
            pltpu.VMEM((6 * LANES,), jnp.uint32),
            pltpu.VMEM((4 * LPB,), jnp.int32),
            pltpu.VMEM((4 * LPB,), jnp.int32),
            pltpu.VMEM((NB * LANES,), jnp.int32),
            pltpu.VMEM((NB * LANES,), jnp.int32),
            pltpu.VMEM((NB * LANES,), jnp.int32),
            pltpu.VMEM((NB * LANES,), jnp.int32),
            pltpu.VMEM((SROWS, LANES), jnp.float32),
            pltpu.VMEM((SROWS, LANES), jnp.float32),
            pltpu.VMEM((RPW, DIM), jnp.float32),
            pltpu.SemaphoreType.DMA,
            pltpu.SemaphoreType.DMA,
        ],
        compiler_params=pltpu.CompilerParams(
            needs_layout_passes=False, use_tc_tiling_on_sc=False),
    )(_body)
    return f(xs, tab2d, cf)


def kernel(x, table0, coeffs0, coeffs1):
    xs = x.astype(jnp.uint32)
    tab2d = table0.reshape(TROWS, LANES)

    def split(c):
        a = c[:, 0]
        return jnp.stack([a >> 16, a & 0xFFFF, c[:, 1]])

    cf = jnp.concatenate([split(coeffs0), split(coeffs1)]).astype(jnp.uint32)
    cf = jnp.pad(cf, ((0, 0), (0, LANES - NCH))).reshape(-1)
    return _sc_call(xs, tab2d, cf)


# speedup vs baseline: 1.3017x; 1.0512x over previous
import functools

import jax
import jax.numpy as jnp
from jax import lax
from jax.experimental import pallas as pl
from jax.experimental.pallas import tpu as pltpu
from jax.experimental.pallas import tpu_sc as plsc

B = 16384
DIM = 32
NCH = 8
SIZE = 16777216
LANES = 16
TROWS = SIZE // LANES
RMASK = TROWS - 1
PRIME = (1 << 31) - 1

NC, NS = 2, 16
NW = NC * NS
RPW = B // NW
NB = 16
NBLK = RPW // NB
LPB = NB * NCH


def _fold(s):
    return (s & jnp.uint32(PRIME)) + (s >> 31)


def _rot(n, k):
    low = (n & jnp.uint32((1 << (31 - k)) - 1)) << k
    high = n >> (31 - k)
    return low + high


def _hash_many(x1, x0, cos):
    t3 = [_rot(a1 * x1, 26) for (a1, a0, bb) in cos]
    t0 = [a0 * x0 for (a1, a0, bb) in cos]
    t2 = [_rot(a1 * x0, 16) for (a1, a0, bb) in cos]
    t1 = [_rot(a0 * x1, 10) for (a1, a0, bb) in cos]
    ss = [_fold(a + b) for a, b in zip(t3, t0)]
    ss = [_fold(s + t) for s, t in zip(ss, t2)]
    ss = [_fold(s + t) for s, t in zip(ss, t1)]
    ss = [_fold(s + bb) for s, (a1, a0, bb) in zip(ss, cos)]
    ss = [_fold(s) for s in ss]
    ss = [jnp.where(s >= jnp.uint32(PRIME), s - jnp.uint32(PRIME), s)
          for s in ss]
    return [s & jnp.uint32(SIZE - 1) for s in ss]


SROWS = 4 * LPB
SFLAT = SROWS * LANES


def _body(x_hbm, tab_hbm, cf_hbm, out_hbm,
          xv, cfv, sidxA, sidxB, bvA, bvB, wbA, wbB,
          sstageA, sstageB, outv, semA, semB):
    wid = lax.axis_index("s") * jnp.int32(NC) + lax.axis_index("c")
    base = wid * jnp.int32(RPW)
    pltpu.sync_copy(x_hbm.at[pl.ds(base, RPW)], xv)
    pltpu.sync_copy(cf_hbm, cfv)
    lanes = lax.iota(jnp.int32, LANES)
    lanes512 = lanes * 512
    pairC = (lanes >> 1) * LANES + (lanes & 1) * NCH

    cfr = [cfv[pl.ds(r * LANES, LANES)] for r in range(6)]
    coef = [[cfr[r][j] for r in range(6)] for j in range(NCH)]

    def hash_block(b, sidx, bv, wb):
        xu = xv[pl.ds(b * jnp.int32(NB), NB)]
        x1 = xu >> 10
        x0 = xu & jnp.uint32(1023)
        for jp in range(NCH // 4):
            js = tuple(4 * jp + u for u in range(4))
            cos = []
            for j in js:
                a1_0, a0_0, b_0, a1_1, a0_1, b_1 = coef[j]
                cos.append((a1_0, a0_0, b_0))
                cos.append((a1_1, a0_1, b_1))
            hs = _hash_many(x1, x0, cos)
            for t, j in enumerate(js):
                h0, h1 = hs[2 * t], hs[2 * t + 1]
                r = (h0 >> 4).astype(jnp.int32)
                o0 = (h0 & jnp.uint32(15)).astype(jnp.int32)
                wr = (h1 >> 4).astype(jnp.int32)
                wo = (h1 & jnp.uint32(15)).astype(jnp.int32)
                posb = lanes * 32 + 4 * j
                plsc.store_scatter(sidx, [posb], r)
                plsc.store_scatter(sidx, [posb + 1], (r + 1) & RMASK)
                plsc.store_scatter(sidx, [posb + 2], (r + 2) & RMASK)
                plsc.store_scatter(sidx, [posb + 3], wr)
                slot = pairC + j
                plsc.store_scatter(bv, [slot], lanes512 + (o0 + 64 * j))
                plsc.store_scatter(wb, [slot], lanes512 + (wo + (64 * j + 48)))

    def copies(sidx, sstage, sem):
        return [
            pltpu.make_async_copy(
                tab_hbm.at[sidx.at[pl.ds(d * LPB, LPB)]],
                sstage.at[pl.ds(d * LPB, LPB)], sem)
            for d in range(4)
        ]

    def fire(bufs):
        for c in copies(*bufs):
            c.start()

    def drain(bufs):
        for c in copies(*bufs):
            c.wait()

    NR = 4

    def accum(b, bv, wb, sstage):
        for rp in range(NB // NR):
            rows = tuple(NR * rp + u for u in range(NR))
            bvp, wvp = [], []
            acc = [[None, None, None, None] for _ in rows]
            for q in range(NR // 2):
                pr = rows[2 * q] // 2
                wp = wb[pl.ds(pr * LANES, LANES)]
                bvp.append(bv[pl.ds(pr * LANES, LANES)])
                wvp.append(plsc.load_gather(sstage, [wp >> 4, wp & 15]))
            for j in range(NCH):
                par = j & 1
                for t in range(NR):
                    sl = (t & 1) * NCH + j
                    f0 = bvp[t >> 1][sl] + lanes
                    r0 = f0 >> 4
                    e0 = f0 & 15
                    g0 = plsc.load_gather(sstage, [r0, e0])
                    g1 = plsc.load_gather(sstage, [r0 + 1, e0])
                    w = wvp[t >> 1][sl]
                    p0, p1 = g0 * w, g1 * w
                    a = acc[t]
                    a[par] = p0 if a[par] is None else a[par] + p0
                    a[2 + par] = p1 if a[2 + par] is None else a[2 + par] + p1
            for t in range(NR):
                orow = b * jnp.int32(NB) + rows[t]
                a = acc[t]
                outv[orow, pl.ds(0, LANES)] = (a[0] + a[1]) * 2.0
                outv[orow, pl.ds(LANES, LANES)] = (a[2] + a[3]) * 2.0

    bufsA = (sidxA, sstageA, semA)
    bufsB = (sidxB, sstageB, semB)

    hash_block(jnp.int32(0), sidxA, bvA, wbA)
    fire(bufsA)

    def pair(i, carry):
        b0 = i * jnp.int32(2)
        b1 = b0 + 1
        hash_block(b1, sidxB, bvB, wbB)
        fire(bufsB)
        drain(bufsA)
        accum(b0, bvA, wbA, sstageA)

        @pl.when(i < jnp.int32(NBLK // 2 - 1))
        def _():
            hash_block(b0 + 2, sidxA, bvA, wbA)
            fire(bufsA)

        drain(bufsB)
        accum(b1, bvB, wbB, sstageB)
        return carry

    lax.fori_loop(jnp.int32(0), jnp.int32(NBLK // 2), pair, jnp.int32(0))
    pltpu.sync_copy(outv, out_hbm.at[pl.ds(base, RPW)])


@jax.jit
def _sc_call(xs, tab2d, cf):
    mesh = plsc.VectorSubcoreMesh(core_axis_name="c", subcore_axis_name="s")
    f = functools.partial(
        pl.kernel,
        out_type=jax.ShapeDtypeStruct((B, DIM), jnp.float32),
        mesh=mesh,
        scratch_types=[
            pltpu.VMEM((RPW,), jnp.uint32),
            pltpu.VMEM((6 * LANES,), jnp.uint32),
            pltpu.VMEM((4 * LPB,), jnp.int32),
            pltpu.VMEM((4 * LPB,), jnp.int32),
            pltpu.VMEM((NB * LANES // 2,), jnp.int32),
            pltpu.VMEM((NB * LANES // 2,), jnp.int32),
            pltpu.VMEM((NB * LANES // 2,), jnp.int32),
            pltpu.VMEM((NB * LANES // 2,), jnp.int32),
            pltpu.VMEM((SROWS, LANES), jnp.float32),
            pltpu.VMEM((SROWS, LANES), jnp.float32),
            pltpu.VMEM((RPW, DIM), jnp.float32),
            pltpu.SemaphoreType.DMA,
            pltpu.SemaphoreType.DMA,
        ],
        compiler_params=pltpu.CompilerParams(
            needs_layout_passes=False, use_tc_tiling_on_sc=False),
    )(_body)
    return f(xs, tab2d, cf)


def kernel(x, table0, coeffs0, coeffs1):
    xs = x.astype(jnp.uint32)
    tab2d = table0.reshape(TROWS, LANES)

    def split(c):
        a = c[:, 0]
        return jnp.stack([a >> 16, a & 0xFFFF, c[:, 1]])

    cf = jnp.concatenate([split(coeffs0), split(coeffs1)]).astype(jnp.uint32)
    cf = jnp.pad(cf, ((0, 0), (0, LANES - NCH))).reshape(-1)
    return _sc_call(xs, tab2d, cf)


# speedup vs baseline: 1.3062x; 1.0034x over previous
import functools

import jax
import jax.numpy as jnp
from jax import lax
from jax.experimental import pallas as pl
from jax.experimental.pallas import tpu as pltpu
from jax.experimental.pallas import tpu_sc as plsc

B = 16384
DIM = 32
NCH = 8
SIZE = 16777216
LANES = 16
TROWS = SIZE // LANES
RMASK = TROWS - 1
PRIME = (1 << 31) - 1

NC, NS = 2, 16
NW = NC * NS
RPW = B // NW
NB = 16
NBLK = RPW // NB
LPB = NB * NCH


def _fold(s):
    return (s & jnp.uint32(PRIME)) + (s >> 31)


def _rot(n, k):
    low = (n & jnp.uint32((1 << (31 - k)) - 1)) << k
    high = n >> (31 - k)
    return low + high


def _hash_many(x1, x0, cos):
    t3 = [_rot(a1 * x1, 26) for (a1, a0, bb) in cos]
    t0 = [a0 * x0 for (a1, a0, bb) in cos]
    t2 = [_rot(a1 * x0, 16) for (a1, a0, bb) in cos]
    t1 = [_rot(a0 * x1, 10) for (a1, a0, bb) in cos]
    ss = [_fold(a + b) for a, b in zip(t3, t0)]
    ss = [_fold(s + t) for s, t in zip(ss, t2)]
    ss = [_fold(s + t) for s, t in zip(ss, t1)]
    ss = [_fold(s + bb) for s, (a1, a0, bb) in zip(ss, cos)]
    ss = [_fold(s) for s in ss]
    ss = [jnp.where(s >= jnp.uint32(PRIME), s - jnp.uint32(PRIME), s)
          for s in ss]
    return [s & jnp.uint32(SIZE - 1) for s in ss]


SROWS = 4 * LPB
SFLAT = SROWS * LANES


def _body(x_hbm, tab_hbm, cf_hbm, out_hbm,
          xv, cfv, sidxA, sidxB, bvA, bvB, wbA, wbB,
          sstageA, sstageB, outv, semA, semB):
    wid = lax.axis_index("s") * jnp.int32(NC) + lax.axis_index("c")
    base = wid * jnp.int32(RPW)
    pltpu.sync_copy(x_hbm.at[pl.ds(base, RPW)], xv)
    pltpu.sync_copy(cf_hbm, cfv)
    lanes = lax.iota(jnp.int32, LANES)
    lanes512 = lanes * 512
    pairC = (lanes >> 1) * LANES + (lanes & 1) * NCH

    cfr = [cfv[pl.ds(r * LANES, LANES)] for r in range(6)]
    coef = [[cfr[r][j] for r in range(6)] for j in range(NCH)]

    def hash_block(b, sidx, bv, wb):
        xu = xv[pl.ds(b * jnp.int32(NB), NB)]
        x1 = xu >> 10
        x0 = xu & jnp.uint32(1023)
        for jp in range(NCH // 4):
            js = tuple(4 * jp + u for u in range(4))
            cos = []
            for j in js:
                a1_0, a0_0, b_0, a1_1, a0_1, b_1 = coef[j]
                cos.append((a1_0, a0_0, b_0))
                cos.append((a1_1, a0_1, b_1))
            hs = _hash_many(x1, x0, cos)
            for t, j in enumerate(js):
                h0, h1 = hs[2 * t], hs[2 * t + 1]
                r = (h0 >> 4).astype(jnp.int32)
                o0 = (h0 & jnp.uint32(15)).astype(jnp.int32)
                wr = (h1 >> 4).astype(jnp.int32)
                wo = (h1 & jnp.uint32(15)).astype(jnp.int32)
                posb = lanes * 32 + 4 * j
                plsc.store_scatter(sidx, [posb], r)
                plsc.store_scatter(sidx, [posb + 1], (r + 1) & RMASK)
                plsc.store_scatter(sidx, [posb + 2], (r + 2) & RMASK)
                plsc.store_scatter(sidx, [posb + 3], wr)
                slot = pairC + j
                plsc.store_scatter(bv, [slot], lanes512 + (o0 + 64 * j))
                plsc.store_scatter(wb, [slot], lanes512 + (wo + (64 * j + 48)))

    def copies(sidx, sstage, sem):
        return [
            pltpu.make_async_copy(
                tab_hbm.at[sidx.at[pl.ds(d * LPB, LPB)]],
                sstage.at[pl.ds(d * LPB, LPB)], sem)
            for d in range(4)
        ]

    def fire(bufs):
        for c in copies(*bufs):
            c.start()

    def drain(bufs):
        for c in copies(*bufs):
            c.wait()

    NR = 8

    def accum(b, bv, wb, sstage):
        for rp in range(NB // NR):
            rows = tuple(NR * rp + u for u in range(NR))
            bvp, wvp = [], []
            acc = [[None, None, None, None] for _ in rows]
            for q in range(NR // 2):
                pr = rows[2 * q] // 2
                wp = wb[pl.ds(pr * LANES, LANES)]
                bvp.append(bv[pl.ds(pr * LANES, LANES)])
                wvp.append(plsc.load_gather(sstage, [wp >> 4, wp & 15]))
            for j in range(NCH):
                par = j & 1
                for t in range(NR):
                    sl = (t & 1) * NCH + j
                    f0 = bvp[t >> 1][sl] + lanes
                    r0 = f0 >> 4
                    e0 = f0 & 15
                    g0 = plsc.load_gather(sstage, [r0, e0])
                    g1 = plsc.load_gather(sstage, [r0 + 1, e0])
                    w = wvp[t >> 1][sl]
                    p0, p1 = g0 * w, g1 * w
                    a = acc[t]
                    a[par] = p0 if a[par] is None else a[par] + p0
                    a[2 + par] = p1 if a[2 + par] is None else a[2 + par] + p1
            for t in range(NR):
                orow = b * jnp.int32(NB) + rows[t]
                a = acc[t]
                outv[orow, pl.ds(0, LANES)] = (a[0] + a[1]) * 2.0
                outv[orow, pl.ds(LANES, LANES)] = (a[2] + a[3]) * 2.0

    bufsA = (sidxA, sstageA, semA)
    bufsB = (sidxB, sstageB, semB)

    hash_block(jnp.int32(0), sidxA, bvA, wbA)
    fire(bufsA)

    def pair(i, carry):
        b0 = i * jnp.int32(2)
        b1 = b0 + 1
        hash_block(b1, sidxB, bvB, wbB)
        fire(bufsB)
        drain(bufsA)
        accum(b0, bvA, wbA, sstageA)

        @pl.when(i < jnp.int32(NBLK // 2 - 1))
        def _():
            hash_block(b0 + 2, sidxA, bvA, wbA)
            fire(bufsA)

        drain(bufsB)
        accum(b1, bvB, wbB, sstageB)
        return carry

    lax.fori_loop(jnp.int32(0), jnp.int32(NBLK // 2), pair, jnp.int32(0))
    pltpu.sync_copy(outv, out_hbm.at[pl.ds(base, RPW)])


@jax.jit
def _sc_call(xs, tab2d, cf):
    mesh = plsc.VectorSubcoreMesh(core_axis_name="c", subcore_axis_name="s")
    f = functools.partial(
        pl.kernel,
        out_type=jax.ShapeDtypeStruct((B, DIM), jnp.float32),
        mesh=mesh,
        scratch_types=[
            pltpu.VMEM((RPW,), jnp.uint32),
            pltpu.VMEM((6 * LANES,), jnp.uint32),
            pltpu.VMEM((4 * LPB,), jnp.int32),
            pltpu.VMEM((4 * LPB,), jnp.int32),
            pltpu.VMEM((NB * LANES // 2,), jnp.int32),
            pltpu.VMEM((NB * LANES // 2,), jnp.int32),
            pltpu.VMEM((NB * LANES // 2,), jnp.int32),
            pltpu.VMEM((NB * LANES // 2,), jnp.int32),
            pltpu.VMEM((SROWS, LANES), jnp.float32),
            pltpu.VMEM((SROWS, LANES), jnp.float32),
            pltpu.VMEM((RPW, DIM), jnp.float32),
            pltpu.SemaphoreType.DMA,
            pltpu.SemaphoreType.DMA,
        ],
        compiler_params=pltpu.CompilerParams(
            needs_layout_passes=False, use_tc_tiling_on_sc=False),
    )(_body)
    return f(xs, tab2d, cf)


def kernel(x, table0, coeffs0, coeffs1):
    xs = x.astype(jnp.uint32)
    tab2d = table0.reshape(TROWS, LANES)

    def split(c):
        a = c[:, 0]
        return jnp.stack([a >> 16, a & 0xFFFF, c[:, 1]])

    cf = jnp.concatenate([split(coeffs0), split(coeffs1)]).astype(jnp.uint32)
    cf = jnp.pad(cf, ((0, 0), (0, LANES - NCH))).reshape(-1)
    return _sc_call(xs, tab2d, cf)


# speedup vs baseline: 1.3093x; 1.0023x over previous
import functools

import jax
import jax.numpy as jnp
from jax import lax
from jax.experimental import pallas as pl
from jax.experimental.pallas import tpu as pltpu
from jax.experimental.pallas import tpu_sc as plsc

B = 16384
DIM = 32
NCH = 8
SIZE = 16777216
LANES = 16
TROWS = SIZE // LANES
RMASK = TROWS - 1
PRIME = (1 << 31) - 1

NC, NS = 2, 16
NW = NC * NS
RPW = B // NW
NB = 16
NBLK = RPW // NB
LPB = NB * NCH


def _fold(s):
    return (s & jnp.uint32(PRIME)) + (s >> 31)


def _rot(n, k):
    low = (n & jnp.uint32((1 << (31 - k)) - 1)) << k
    high = n >> (31 - k)
    return low + high


def _hash_many(x1, x0, cos):
    t3 = [_rot(a1 * x1, 26) for (a1, a0, bb) in cos]
    t0 = [a0 * x0 for (a1, a0, bb) in cos]
    t2 = [_rot(a1 * x0, 16) for (a1, a0, bb) in cos]
    t1 = [_rot(a0 * x1, 10) for (a1, a0, bb) in cos]
    ss = [_fold(a + b) for a, b in zip(t3, t0)]
    ss = [_fold(s + t) for s, t in zip(ss, t2)]
    ss = [_fold(s + t) for s, t in zip(ss, t1)]
    ss = [_fold(s + bb) for s, (a1, a0, bb) in zip(ss, cos)]
    ss = [_fold(s) for s in ss]
    ss = [jnp.where(s >= jnp.uint32(PRIME), s - jnp.uint32(PRIME), s)
          for s in ss]
    return [s & jnp.uint32(SIZE - 1) for s in ss]


SROWS = 4 * LPB


def _body(x_hbm, tab_hbm, cf_hbm, out_hbm,
          xv, cfv, sidxA, sidxB, bvA, bvB, wbA, wbB,
          sstageA, sstageB, outv, semA, semB):
    wid = lax.axis_index("s") * jnp.int32(NC) + lax.axis_index("c")
    base = wid * jnp.int32(RPW)
    pltpu.sync_copy(x_hbm.at[pl.ds(base, RPW)], xv)
    pltpu.sync_copy(cf_hbm, cfv)
    lanes = lax.iota(jnp.int32, LANES)
    lanes512 = lanes * 512
    pairC = (lanes >> 1) * LANES + (lanes & 1) * NCH

    cfr = [cfv[pl.ds(r * LANES, LANES)] for r in range(6)]
    coef = [[cfr[r][j] for r in range(6)] for j in range(NCH)]

    def hash_block(b, sidx, bv, wb):
        xu = xv[pl.ds(b * jnp.int32(NB), NB)]
        x1 = xu >> 10
        x0 = xu & jnp.uint32(1023)
        for jp in range(NCH // 4):
            js = tuple(4 * jp + u for u in range(4))
            cos = []
            for j in js:
                a1_0, a0_0, b_0, a1_1, a0_1, b_1 = coef[j]
                cos.append((a1_0, a0_0, b_0))
                cos.append((a1_1, a0_1, b_1))
            hs = _hash_many(x1, x0, cos)
            for t, j in enumerate(js):
                h0, h1 = hs[2 * t], hs[2 * t + 1]
                r = (h0 >> 4).astype(jnp.int32)
                o0 = (h0 & jnp.uint32(15)).astype(jnp.int32)
                wr = (h1 >> 4).astype(jnp.int32)
                wo = (h1 & jnp.uint32(15)).astype(jnp.int32)
                posb = lanes * 32 + 4 * j
                plsc.store_scatter(sidx, [posb], r)
                plsc.store_scatter(sidx, [posb + 1], (r + 1) & RMASK)
                plsc.store_scatter(sidx, [posb + 2], (r + 2) & RMASK)
                plsc.store_scatter(sidx, [posb + 3], wr)
                slot = pairC + j
                plsc.store_scatter(bv, [slot], lanes512 + (o0 + 64 * j))
                plsc.store_scatter(wb, [slot], lanes512 + (wo + (64 * j + 48)))

    def copies(sidx, sstage, sem):
        return [
            pltpu.make_async_copy(
                tab_hbm.at[sidx.at[pl.ds(d * LPB, LPB)]],
                sstage.at[pl.ds(d * LPB, LPB)], sem)
            for d in range(4)
        ]

    def fire(bufs):
        for c in copies(*bufs):
            c.start()

    def drain(bufs):
        for c in copies(*bufs):
            c.wait()

    NR = 8

    def accum(b, bv, wb, sstage):
        for rp in range(NB // NR):
            rows = tuple(NR * rp + u for u in range(NR))
            bvp, wvp = [], []
            acc = [[None, None, None, None] for _ in rows]
            for q in range(NR // 2):
                pr = rows[2 * q] // 2
                wp = wb[pl.ds(pr * LANES, LANES)]
                bvp.append(bv[pl.ds(pr * LANES, LANES)])
                wvp.append(plsc.load_gather(sstage, [wp >> 4, wp & 15]))
            for j in range(NCH):
                par = j & 1
                for t in range(NR):
                    sl = (t & 1) * NCH + j
                    f0 = bvp[t >> 1][sl] + lanes
                    r0 = f0 >> 4
                    e0 = f0 & 15
                    g0 = plsc.load_gather(sstage, [r0, e0])
                    g1 = plsc.load_gather(sstage, [r0 + 1, e0])
                    w = wvp[t >> 1][sl]
                    p0, p1 = g0 * w, g1 * w
                    a = acc[t]
                    a[par] = p0 if a[par] is None else a[par] + p0
                    a[2 + par] = p1 if a[2 + par] is None else a[2 + par] + p1
            for t in range(NR):
                orow = b * jnp.int32(NB) + rows[t]
                a = acc[t]
                outv[orow, pl.ds(0, LANES)] = (a[0] + a[1]) * 2.0
                outv[orow, pl.ds(LANES, LANES)] = (a[2] + a[3]) * 2.0

    bufsA = (sidxA, sstageA, semA)
    bufsB = (sidxB, sstageB, semB)

    hash_block(jnp.int32(0), sidxA, bvA, wbA)
    fire(bufsA)

    def pair(i, carry):
        b0 = i * jnp.int32(2)
        b1 = b0 + 1
        hash_block(b1, sidxB, bvB, wbB)
        fire(bufsB)
        drain(bufsA)
        accum(b0, bvA, wbA, sstageA)

        @pl.when(i < jnp.int32(NBLK // 2 - 1))
        def _():
            hash_block(b0 + 2, sidxA, bvA, wbA)
            fire(bufsA)

        drain(bufsB)
        accum(b1, bvB, wbB, sstageB)
        return carry

    lax.fori_loop(jnp.int32(0), jnp.int32(NBLK // 2), pair, jnp.int32(0))
    pltpu.sync_copy(outv, out_hbm.at[pl.ds(base, RPW)])


@jax.jit
def _sc_call(xs, tab2d, cf):
    mesh = plsc.VectorSubcoreMesh(core_axis_name="c", subcore_axis_name="s")
    f = functools.partial(
        pl.kernel,
        out_type=jax.ShapeDtypeStruct((B, DIM), jnp.float32),
        mesh=mesh,
        scratch_types=[
            pltpu.VMEM((RPW,), jnp.uint32),
            pltpu.VMEM((6 * LANES,), jnp.uint32),
            pltpu.VMEM((4 * LPB,), jnp.int32),
            pltpu.VMEM((4 * LPB,), jnp.int32),
            pltpu.VMEM((NB * LANES // 2,), jnp.int32),
            pltpu.VMEM((NB * LANES // 2,), jnp.int32),
            pltpu.VMEM((NB * LANES // 2,), jnp.int32),
            pltpu.VMEM((NB * LANES // 2,), jnp.int32),
            pltpu.VMEM((SROWS, LANES), jnp.float32),
            pltpu.VMEM((SROWS, LANES), jnp.float32),
            pltpu.VMEM((RPW, DIM), jnp.float32),
            pltpu.SemaphoreType.DMA,
            pltpu.SemaphoreType.DMA,
        ],
        compiler_params=pltpu.CompilerParams(
            needs_layout_passes=False, use_tc_tiling_on_sc=False),
    )(_body)
    return f(xs, tab2d, cf)


def kernel(x, table0, coeffs0, coeffs1):
    xs = x.astype(jnp.uint32)
    tab2d = table0.reshape(TROWS, LANES)

    def split(c):
        a = c[:, 0]
        return jnp.stack([a >> 16, a & 0xFFFF, c[:, 1]])

    cf = jnp.concatenate([split(coeffs0), split(coeffs1)]).astype(jnp.uint32)
    cf = jnp.pad(cf, ((0, 0), (0, LANES - NCH))).reshape(-1)
    return _sc_call(xs, tab2d, cf)
